# baseline (speedup 1.0000x reference)
import functools

import jax
import jax.numpy as jnp
from jax import lax
from jax.experimental import pallas as pl
from jax.experimental.pallas import tpu as pltpu
from jax.experimental.pallas import tpu_sc as plsc

N_POINTS = 1_000_000
EMB = 16
OUT_D = EMB + 3
CH = 800
NCHUNK = N_POINTS // CH
NW = 32
MAX_CH_W = -(-NCHUNK // NW)
NBLK = CH // 16

_mesh = plsc.VectorSubcoreMesh(core_axis_name="c", subcore_axis_name="s")

_BCAST_DNUMS = lax.GatherDimensionNumbers(
    offset_dims=(), collapsed_slice_dims=(0,), start_index_map=(0,))


def _lane_bcast(vec, idx_splat):
    return lax.gather(vec, idx_splat[:, None], _BCAST_DNUMS, (1,),
                      mode=lax.GatherScatterMode.PROMISE_IN_BOUNDS)


@functools.partial(
    pl.kernel,
    mesh=_mesh,
    compiler_params=pltpu.CompilerParams(
        needs_layout_passes=False, use_tc_tiling_on_sc=False),
    out_type=jax.ShapeDtypeStruct((N_POINTS, OUT_D), jnp.float32),
    scratch_types=(
        [pltpu.VMEM((CH,), jnp.int32)] * 2
        + [pltpu.VMEM((CH,), jnp.float32)] * 12
        + [pltpu.VMEM((CH,), jnp.int32)] * 6
        + [pltpu.VMEM((CH, EMB), jnp.float32)] * 6
        + [pltpu.VMEM((CH, OUT_D), jnp.float32)] * 1
        + [pltpu.SemaphoreType.DMA] * 4
    ),
)
def _encode(fidx_hbm, f0_hbm, f1_hbm, f2_hbm, b0_hbm, b1_hbm, b2_hbm,
            x0_hbm, x1_hbm, x2_hbm, emb_hbm, out_hbm,
            fx0, fx1,
            b00, b10, b20, x00, x10, x20,
            b01, b11, b21, x01, x11, x21,
            v00, v10, v20, v01, v11, v21,
            e00, e10, e20, e01, e11, e21,
            ov0, sem_f, sem_x, sem_b, sem_e):
    fxs = (fx0, fx1)
    bxs = ((b00, b10, b20, x00, x10, x20),
           (b01, b11, b21, x01, x11, x21))
    vss = ((v00, v10, v20), (v01, v11, v21))
    es = ((e00, e10, e20), (e01, e11, e21))
    fhbms = (f0_hbm, f1_hbm, f2_hbm)
    bxhbms = (b0_hbm, b1_hbm, b2_hbm, x0_hbm, x1_hbm, x2_hbm)

    wid = lax.axis_index("s") * 2 + lax.axis_index("c")
    nj = jnp.where(wid < NCHUNK - (MAX_CH_W - 1) * NW, MAX_CH_W, MAX_CH_W - 1)

    def issue_f(j, s):
        sl = pl.ds((j * NW + wid) * CH, CH)
        return pltpu.async_copy(fidx_hbm.at[sl], fxs[s], sem_f)

    def drain_f(s):
        pltpu.make_async_copy(fidx_hbm.at[pl.ds(0, CH)], fxs[s], sem_f).wait()

    def issue_bx(j, s):
        sl = pl.ds((j * NW + wid) * CH, CH)
        return [pltpu.async_copy(h.at[sl], d, sem_x)
                for h, d in zip(bxhbms, bxs[s])]

    def drain_bx(s):
        for h, d in zip(bxhbms, bxs[s]):
            pltpu.make_async_copy(h.at[pl.ds(0, CH)], d, sem_x).wait()

    def issue_b(fs, vset):
        return [pltpu.async_copy(h.at[fxs[fs]], d, sem_b)
                for h, d in zip(fhbms, vss[vset])]

    def drain_b(vset):
        for h, d in zip(fhbms, vss[vset]):
            pltpu.make_async_copy(h.at[pl.ds(0, CH)], d, sem_b).wait()

    def issue_c(vset, eset):
        return [pltpu.async_copy(emb_hbm.at[v], e, sem_e)
                for v, e in zip(vss[vset], es[eset])]

    def drain_c(eset):
        for e in es[eset]:
            pltpu.make_async_copy(emb_hbm.at[pl.ds(0, CH)], e, sem_e).wait()

    def compute(j, p):
        e0_v, e1_v, e2_v = es[p]
        b0_v, b1_v, b2_v, x0_v, x1_v, x2_v = bxs[p]
        out_v = ov0

        def blk(k, carry2):
            ksl = pl.ds(k * 16, 16)
            w0 = b0_v[ksl]
            w1 = b1_v[ksl]
            w2 = b2_v[ksl]
            k16 = k * 16
            for pt in range(16):
                i = k16 + pt
                psplat = jnp.full((16,), pt, jnp.int32)
                s0 = _lane_bcast(w0, psplat)
                s1 = _lane_bcast(w1, psplat)
                s2 = _lane_bcast(w2, psplat)
                acc = s0 * e0_v[i] + s1 * e1_v[i] + s2 * e2_v[i]
                out_v[i, pl.ds(0, EMB)] = acc
            rows = lax.iota(jnp.int32, 16) + k16
            for dd, xv in ((EMB, x0_v), (EMB + 1, x1_v), (EMB + 2, x2_v)):
                plsc.store_scatter(
                    out_v, [rows, jnp.full((16,), dd, jnp.int32)], xv[ksl])
            return carry2

        lax.fori_loop(0, NBLK, blk, 0, unroll=2)
        pltpu.sync_copy(out_v, out_hbm.at[pl.ds((j * NW + wid) * CH, CH)])

    issue_f(0, 0).wait()
    for cp in issue_b(0, 0):
        cp.wait()
    issue_c(0, 0)
    issue_f(1, 1).wait()
    issue_b(1, 1)
    for cp in issue_bx(0, 0):
        cp.wait()
    issue_f(2, 0)

    def body(i, carry):
        for q in (0, 1):
            j = i * 2 + q
            p = q
            np_ = 1 - q

            @pl.when(j < nj)
            def _():
                drain_c(p)

                @pl.when(j + 1 < nj)
                def _():
                    drain_b(np_)
                    issue_c(np_, np_)

                @pl.when(j + 2 < nj)
                def _():
                    drain_f(p)
                    issue_b(p, p)

                @pl.when(j + 3 < nj)
                def _():
                    issue_f(j + 3, np_)

                @pl.when(j + 1 < nj)
                def _():
                    issue_bx(j + 1, np_)

                @pl.when(j > 0)
                def _():
                    drain_bx(p)

                compute(j, p)

        return carry

    lax.fori_loop(0, MAX_CH_W // 2, body, 0)


def kernel(x, face_idxs, barycentrics, embeddings, faces):
    return _encode(
        face_idxs,
        faces[:, 0], faces[:, 1], faces[:, 2],
        barycentrics[:, 0], barycentrics[:, 1], barycentrics[:, 2],
        x[:, 0], x[:, 1], x[:, 2],
        embeddings)


# speedup vs baseline: 1.0027x; 1.0027x over previous
import functools

import jax
import jax.numpy as jnp
from jax import lax
from jax.experimental import pallas as pl
from jax.experimental.pallas import tpu as pltpu
from jax.experimental.pallas import tpu_sc as plsc

N_POINTS = 1_000_000
EMB = 16
OUT_D = EMB + 3
CH = 800
NCHUNK = N_POINTS // CH
NW = 32
MAX_CH_W = -(-NCHUNK // NW)
NBLK = CH // 16

_mesh = plsc.VectorSubcoreMesh(core_axis_name="c", subcore_axis_name="s")

_BCAST_DNUMS = lax.GatherDimensionNumbers(
    offset_dims=(), collapsed_slice_dims=(0,), start_index_map=(0,))


def _lane_bcast(vec, idx_splat):
    return lax.gather(vec, idx_splat[:, None], _BCAST_DNUMS, (1,),
                      mode=lax.GatherScatterMode.PROMISE_IN_BOUNDS)


@functools.partial(
    pl.kernel,
    mesh=_mesh,
    compiler_params=pltpu.CompilerParams(
        needs_layout_passes=False, use_tc_tiling_on_sc=False),
    out_type=jax.ShapeDtypeStruct((N_POINTS, OUT_D), jnp.float32),
    scratch_types=(
        [pltpu.VMEM((CH,), jnp.int32)] * 2
        + [pltpu.VMEM((CH,), jnp.float32)] * 12
        + [pltpu.VMEM((CH,), jnp.int32)] * 6
        + [pltpu.VMEM((CH, EMB), jnp.float32)] * 6
        + [pltpu.VMEM((CH, OUT_D), jnp.float32)] * 1
        + [pltpu.SemaphoreType.DMA] * 5
    ),
)
def _encode(fidx_hbm, f0_hbm, f1_hbm, f2_hbm, b0_hbm, b1_hbm, b2_hbm,
            x0_hbm, x1_hbm, x2_hbm, emb_hbm, out_hbm,
            fx0, fx1,
            b00, b10, b20, x00, x10, x20,
            b01, b11, b21, x01, x11, x21,
            v00, v10, v20, v01, v11, v21,
            e00, e10, e20, e01, e11, e21,
            ov0, sem_f, sem_x, sem_b, sem_e, sem_o):
    fxs = (fx0, fx1)
    bxs = ((b00, b10, b20, x00, x10, x20),
           (b01, b11, b21, x01, x11, x21))
    vss = ((v00, v10, v20), (v01, v11, v21))
    es = ((e00, e10, e20), (e01, e11, e21))
    fhbms = (f0_hbm, f1_hbm, f2_hbm)
    bxhbms = (b0_hbm, b1_hbm, b2_hbm, x0_hbm, x1_hbm, x2_hbm)

    wid = lax.axis_index("s") * 2 + lax.axis_index("c")
    nj = jnp.where(wid < NCHUNK - (MAX_CH_W - 1) * NW, MAX_CH_W, MAX_CH_W - 1)

    def issue_f(j, s):
        sl = pl.ds((j * NW + wid) * CH, CH)
        return pltpu.async_copy(fidx_hbm.at[sl], fxs[s], sem_f)

    def drain_f(s):
        pltpu.make_async_copy(fidx_hbm.at[pl.ds(0, CH)], fxs[s], sem_f).wait()

    def issue_bx(j, s):
        sl = pl.ds((j * NW + wid) * CH, CH)
        return [pltpu.async_copy(h.at[sl], d, sem_x)
                for h, d in zip(bxhbms, bxs[s])]

    def drain_bx(s):
        for h, d in zip(bxhbms, bxs[s]):
            pltpu.make_async_copy(h.at[pl.ds(0, CH)], d, sem_x).wait()

    def issue_b(fs, vset):
        return [pltpu.async_copy(h.at[fxs[fs]], d, sem_b)
                for h, d in zip(fhbms, vss[vset])]

    def drain_b(vset):
        for h, d in zip(fhbms, vss[vset]):
            pltpu.make_async_copy(h.at[pl.ds(0, CH)], d, sem_b).wait()

    def issue_c(vset, eset):
        return [pltpu.async_copy(emb_hbm.at[v], e, sem_e)
                for v, e in zip(vss[vset], es[eset])]

    def drain_c(eset):
        for e in es[eset]:
            pltpu.make_async_copy(emb_hbm.at[pl.ds(0, CH)], e, sem_e).wait()

    def drain_o():
        pltpu.make_async_copy(ov0, out_hbm.at[pl.ds(0, CH)], sem_o).wait()

    def compute(j, p):
        e0_v, e1_v, e2_v = es[p]
        b0_v, b1_v, b2_v, x0_v, x1_v, x2_v = bxs[p]
        out_v = ov0

        def blk(k, carry2):
            ksl = pl.ds(k * 16, 16)
            w0 = b0_v[ksl]
            w1 = b1_v[ksl]
            w2 = b2_v[ksl]
            k16 = k * 16
            for pt in range(16):
                i = k16 + pt
                psplat = jnp.full((16,), pt, jnp.int32)
                s0 = _lane_bcast(w0, psplat)
                s1 = _lane_bcast(w1, psplat)
                s2 = _lane_bcast(w2, psplat)
                acc = s0 * e0_v[i] + s1 * e1_v[i] + s2 * e2_v[i]
                out_v[i, pl.ds(0, EMB)] = acc
            rows = lax.iota(jnp.int32, 16) + k16
            for dd, xv in ((EMB, x0_v), (EMB + 1, x1_v), (EMB + 2, x2_v)):
                plsc.store_scatter(
                    out_v, [rows, jnp.full((16,), dd, jnp.int32)], xv[ksl])
            return carry2

        lax.fori_loop(0, NBLK, blk, 0, unroll=2)
        pltpu.async_copy(
            out_v, out_hbm.at[pl.ds((j * NW + wid) * CH, CH)], sem_o)

    issue_f(0, 0).wait()
    for cp in issue_b(0, 0):
        cp.wait()
    issue_c(0, 0)
    issue_f(1, 1).wait()
    issue_b(1, 1)
    for cp in issue_bx(0, 0):
        cp.wait()
    issue_f(2, 0)

    def body(i, carry):
        for q in (0, 1):
            j = i * 2 + q
            p = q
            np_ = 1 - q

            @pl.when(j < nj)
            def _():
                drain_c(p)

                @pl.when(j + 1 < nj)
                def _():
                    drain_b(np_)
                    issue_c(np_, np_)

                @pl.when(j + 2 < nj)
                def _():
                    drain_f(p)
                    issue_b(p, p)

                @pl.when(j + 3 < nj)
                def _():
                    issue_f(j + 3, np_)

                @pl.when(j + 1 < nj)
                def _():
                    issue_bx(j + 1, np_)

                @pl.when(j > 0)
                def _():
                    drain_bx(p)
                    drain_o()

                compute(j, p)

        return carry

    lax.fori_loop(0, MAX_CH_W // 2, body, 0)
    drain_o()


def kernel(x, face_idxs, barycentrics, embeddings, faces):
    return _encode(
        face_idxs,
        faces[:, 0], faces[:, 1], faces[:, 2],
        barycentrics[:, 0], barycentrics[:, 1], barycentrics[:, 2],
        x[:, 0], x[:, 1], x[:, 2],
        embeddings)
